# baseline (speedup 1.0000x reference)
import functools

import jax
import jax.numpy as jnp
from jax import lax
from jax.experimental import pallas as pl
from jax.experimental.pallas import tpu as pltpu
from jax.experimental.pallas import tpu_sc as plsc

N_FIELDS = 26
VOCAB = 100000
EMB_DIM = 32
BATCH = 16384

_NC, _NS = 2, 16
_NW = _NC * _NS
_R = BATCH * N_FIELDS
_RPW = _R // _NW
_CH = 1024
_NCHUNK = _RPW // _CH
_KG = _CH // 128
_NBUF = 3

_VB = 800
_NVB = VOCAB // _VB


def _tc_repack(t832):
  def body(in_ref, out_ref):
    eye = jnp.eye(EMB_DIM, dtype=jnp.float32)
    for d in range(4):
      q = VOCAB // 4
      x = in_ref[:, d * q:(d + 1) * q]
      out_ref[:, d * EMB_DIM:(d + 1) * EMB_DIM] = jax.lax.dot_general(
          x, eye, dimension_numbers=(((0,), (0,)), ((), ())),
          preferred_element_type=jnp.float32)

  return pl.pallas_call(
      body,
      grid=(N_FIELDS,),
      compiler_params=pltpu.CompilerParams(
          vmem_limit_bytes=100 * 1024 * 1024),
      in_specs=[pl.BlockSpec((EMB_DIM, VOCAB), lambda i: (i, 0))],
      out_specs=pl.BlockSpec((VOCAB // 4, 128), lambda i: (i, 0)),
      out_shape=jax.ShapeDtypeStruct((N_FIELDS * VOCAB // 4, 128),
                                     jnp.float32),
  )(t832)


def _sc_gather(x2d, flat_table):
  mesh = plsc.VectorSubcoreMesh(core_axis_name="c", subcore_axis_name="s")

  @functools.partial(
      pl.kernel,
      mesh=mesh,
      compiler_params=pltpu.CompilerParams(use_tc_tiling_on_sc=False),
      out_type=jax.ShapeDtypeStruct((_R, EMB_DIM), jnp.float32),
      scratch_types=[
          pltpu.VMEM((_NBUF, _KG, 128), jnp.int32),
          pltpu.VMEM((_NBUF, _KG, 128), jnp.int32),
          pltpu.VMEM((_NBUF, _CH, EMB_DIM), jnp.float32),
          pltpu.SemaphoreType.DMA((_NBUF,)),
          pltpu.SemaphoreType.DMA((_NBUF,)),
      ],
  )
  def k(x_hbm, tab_hbm, out_hbm, xv, iv, rows, gsem, wsem):
    wid = lax.axis_index("s") * _NC + lax.axis_index("c")

    g_descs = {}
    wb_descs = {}

    def start_chunk(c):
      b = c % _NBUF
      pltpu.sync_copy(x_hbm.at[pl.ds(wid * (_RPW // 128) + c * _KG, _KG)],
                      xv.at[b])
      lanes = lax.iota(jnp.int32, 16)

      def grp(t, carry):
        j = lax.div(t, 8)
        o = lax.rem(t, 8)
        s = pl.ds(o * 16, 16)
        rbase = c * _CH + t * 16
        fld = lax.rem(rbase + lanes, N_FIELDS)
        v = xv[b, j, s]
        d = lax.div(v, jnp.int32(VOCAB // 4))
        m = v - d * (VOCAB // 4)
        iv[b, j, s] = fld * VOCAB + m * 4 + d
        return carry

      lax.fori_loop(0, _KG * 8, grp, 0)
      g_descs[c] = [
          pltpu.async_copy(tab_hbm.at[iv.at[b, j]],
                           rows.at[b, pl.ds(j * 128, 128)], gsem.at[b])
          for j in range(_KG)
      ]

    def retire_chunk(c):
      b = c % _NBUF
      for d in g_descs.pop(c):
        d.wait()
      wb_descs[c] = pltpu.async_copy(
          rows.at[b], out_hbm.at[pl.ds(wid * _RPW + c * _CH, _CH)], wsem.at[b])

    for c in range(_NCHUNK):
      if c >= _NBUF:
        wb_descs.pop(c - _NBUF).wait()
      start_chunk(c)
      if c >= 2:
        retire_chunk(c - 2)
    for c in range(_NCHUNK - 2, _NCHUNK):
      retire_chunk(c)
    for c in range(_NCHUNK - _NBUF, _NCHUNK):
      wb_descs.pop(c).wait()

  return k(x2d, flat_table)


def kernel(x, tables):
  t832 = jnp.swapaxes(tables, 1, 2).reshape(N_FIELDS * EMB_DIM, VOCAB)
  tab128 = _tc_repack(t832)
  flat = tab128.reshape(N_FIELDS * VOCAB, EMB_DIM)
  x2d = x.astype(jnp.int32).reshape(_R // 128, 128)
  out = _sc_gather(x2d, flat)
  return out.reshape(BATCH, N_FIELDS * EMB_DIM)


# speedup vs baseline: 1.0580x; 1.0580x over previous
import functools

import jax
import jax.numpy as jnp
from jax import lax
from jax.experimental import pallas as pl
from jax.experimental.pallas import tpu as pltpu
from jax.experimental.pallas import tpu_sc as plsc

N_FIELDS = 26
VOCAB = 100000
EMB_DIM = 32
BATCH = 16384

_NC, _NS = 2, 16
_NW = _NC * _NS
_R = BATCH * N_FIELDS
_RPW = _R // _NW
_CH = 1024
_NCHUNK = _RPW // _CH
_KG = _CH // 128
_NBUF = 3

_VB = 800
_NVB = VOCAB // _VB


def _tc_repack(t832):
  def body(in_ref, out_ref):
    for d in range(4):
      q = VOCAB // 4
      out_ref[:, d * EMB_DIM:(d + 1) * EMB_DIM] = jnp.transpose(
          in_ref[:, d * q:(d + 1) * q])

  return pl.pallas_call(
      body,
      grid=(N_FIELDS,),
      compiler_params=pltpu.CompilerParams(
          vmem_limit_bytes=100 * 1024 * 1024),
      in_specs=[pl.BlockSpec((EMB_DIM, VOCAB), lambda i: (i, 0))],
      out_specs=pl.BlockSpec((VOCAB // 4, 128), lambda i: (i, 0)),
      out_shape=jax.ShapeDtypeStruct((N_FIELDS * VOCAB // 4, 128),
                                     jnp.float32),
  )(t832)


def _sc_gather(x2d, ring, flat_table):
  mesh = plsc.VectorSubcoreMesh(core_axis_name="c", subcore_axis_name="s")

  @functools.partial(
      pl.kernel,
      mesh=mesh,
      compiler_params=pltpu.CompilerParams(use_tc_tiling_on_sc=False),
      out_type=jax.ShapeDtypeStruct((_R, EMB_DIM), jnp.float32),
      scratch_types=[
          pltpu.VMEM((_NBUF, _KG, 128), jnp.int32),
          pltpu.VMEM((_NBUF, _KG, 128), jnp.int32),
          pltpu.VMEM((48,), jnp.int32),
          pltpu.VMEM((_NBUF, _CH, EMB_DIM), jnp.float32),
          pltpu.SemaphoreType.DMA((_NBUF,)),
          pltpu.SemaphoreType.DMA((_NBUF,)),
      ],
  )
  def k(x_hbm, ring_hbm, tab_hbm, out_hbm, xv, iv, rv, rows, gsem, wsem):
    wid = lax.axis_index("s") * _NC + lax.axis_index("c")
    pltpu.sync_copy(ring_hbm, rv)

    g_descs = {}
    wb_descs = {}

    def start_chunk(c):
      b = c % _NBUF
      pltpu.sync_copy(x_hbm.at[pl.ds(wid * (_RPW // 128) + c * _KG, _KG)],
                      xv.at[b])
      for j in range(_KG):
        for o in range(128 // 16):
          s = pl.ds(o * 16, 16)
          ph = (c * _CH + j * 128 + o * 16) % N_FIELDS
          iv[b, j, s] = xv[b, j, s] + rv[pl.ds(ph, 16)]
      g_descs[c] = [
          pltpu.async_copy(tab_hbm.at[iv.at[b, j]],
                           rows.at[b, pl.ds(j * 128, 128)], gsem.at[b])
          for j in range(_KG)
      ]

    def retire_chunk(c):
      b = c % _NBUF
      for d in g_descs.pop(c):
        d.wait()
      wb_descs[c] = pltpu.async_copy(
          rows.at[b], out_hbm.at[pl.ds(wid * _RPW + c * _CH, _CH)], wsem.at[b])

    for c in range(_NCHUNK):
      if c >= _NBUF:
        wb_descs.pop(c - _NBUF).wait()
      start_chunk(c)
      if c >= 2:
        retire_chunk(c - 2)
    for c in range(_NCHUNK - 2, _NCHUNK):
      retire_chunk(c)
    for c in range(_NCHUNK - _NBUF, _NCHUNK):
      wb_descs.pop(c).wait()

  return k(x2d, ring, flat_table)


def kernel(x, tables):
  t832 = jnp.swapaxes(tables, 1, 2).reshape(N_FIELDS * EMB_DIM, VOCAB)
  tab128 = _tc_repack(t832)
  flat = tab128.reshape(N_FIELDS * VOCAB, EMB_DIM)
  xi = x.astype(jnp.int32)
  xm = (xi % (VOCAB // 4)) * 4 + xi // (VOCAB // 4)
  x2d = xm.reshape(_R // 128, 128)
  ring = (jnp.arange(48, dtype=jnp.int32) % N_FIELDS) * VOCAB
  out = _sc_gather(x2d, ring, flat)
  return out.reshape(BATCH, N_FIELDS * EMB_DIM)
